# Initial kernel scaffold; baseline (speedup 1.0000x reference)
import jax
import jax.numpy as jnp
from jax.experimental import pallas as pl


def kernel(obs, std, obs_sensor, W_t, b_t, ln_g, ln_b, W_s1, b_s1, W_s2, b_s2, W_f1, b_f1, W_f2, b_f2, W_p1, b_p1, W_g1, b_g1, W_g2, b_g2, W_e1, b_e1, W_e2, b_e2, W_p2, b_p2):
    raise NotImplementedError("write your pallas kernel here")


# speedup vs baseline: 1.7087x; 1.7087x over previous
import functools

import jax
import jax.numpy as jnp
from jax.experimental import pallas as pl
from jax.experimental.pallas import tpu as pltpu

B = 1024
REPR_DIM = 4096
FEATURE_DIM = 512
HIDDEN_DIM = 1024
STATE_DIM = 64
GATE_DIM = 256
MOE_HIDDEN = 256
NUM_EXPERTS = 32
TOP_K = 4
ACTION_DIM = 12

BB = 256
NB = B // BB


def _k1_body(obs_ref, obs_sensor_ref, W_t_ref, b_t_ref, ln_g_ref, ln_b_ref,
             W_s1_ref, b_s1_ref, W_s2_ref, b_s2_ref, W_f1_ref, b_f1_ref,
             W_f2_ref, b_f2_ref, W_p1_ref, b_p1_ref, W_g1_ref, b_g1_ref,
             W_g2_ref, b_g2_ref,
             x_out_ref, combine_out_ref, aux_out_ref, acc_ref):
    b = pl.program_id(0)
    f32 = jnp.float32

    obs = obs_ref[...]
    h0 = jnp.dot(obs, W_t_ref[...], preferred_element_type=f32) + b_t_ref[...]
    mu = jnp.mean(h0, axis=-1, keepdims=True)
    var = jnp.mean((h0 - mu) ** 2, axis=-1, keepdims=True)
    h = jnp.tanh((h0 - mu) / jnp.sqrt(var + 1e-5) * ln_g_ref[...] + ln_b_ref[...])

    s1 = jax.nn.relu(
        jnp.dot(obs_sensor_ref[...], W_s1_ref[...], preferred_element_type=f32)
        + b_s1_ref[...])
    s = jnp.dot(s1, W_s2_ref[...], preferred_element_type=f32) + b_s2_ref[...]
    h = h + s

    f1 = jax.nn.relu(
        jnp.dot(h, W_f1_ref[...], preferred_element_type=f32) + b_f1_ref[...])
    h = jnp.dot(f1, W_f2_ref[...], preferred_element_type=f32) + b_f2_ref[...]

    x = jax.nn.relu(
        jnp.dot(h, W_p1_ref[...], preferred_element_type=f32) + b_p1_ref[...])
    x_out_ref[...] = x

    g1 = jax.nn.relu(
        jnp.dot(x, W_g1_ref[...], preferred_element_type=f32) + b_g1_ref[...])
    gl = jnp.dot(g1, W_g2_ref[...], preferred_element_type=f32) + b_g2_ref[...]

    m = jnp.max(gl, axis=-1, keepdims=True)
    ex = jnp.exp(gl - m)
    p = ex / jnp.sum(ex, axis=-1, keepdims=True)

    tri = (jax.lax.broadcasted_iota(jnp.int32, (NUM_EXPERTS, NUM_EXPERTS), 0)
           <= jax.lax.broadcasted_iota(jnp.int32, (NUM_EXPERTS, NUM_EXPERTS), 1)
           ).astype(f32)
    rem = p
    sel = jnp.zeros_like(p)
    for _ in range(TOP_K):
        mk = jnp.max(rem, axis=-1, keepdims=True)
        hit = (rem == mk).astype(f32)
        cs = jnp.dot(hit, tri, preferred_element_type=f32)
        first = hit * (cs == 1.0).astype(f32)
        sel = sel + first
        rem = rem - first * (rem + 1.0)
    topsum = jnp.sum(p * sel, axis=-1, keepdims=True)
    combine = p * sel / topsum
    combine_out_ref[...] = combine

    counts = jnp.sum(sel, axis=0, keepdims=True)
    psum = jnp.sum(p, axis=0, keepdims=True)
    part = jnp.concatenate([counts, psum, jnp.zeros((6, NUM_EXPERTS), f32)],
                           axis=0)

    @pl.when(b == 0)
    def _():
        acc_ref[...] = jnp.zeros_like(acc_ref)

    acc_ref[...] += part
    aux = (NUM_EXPERTS / (B * TOP_K * B)) * jnp.sum(
        acc_ref[0, :] * acc_ref[1, :])
    aux_out_ref[...] = jnp.full((8, 128), aux, f32)


def _k2_body(x_ref, combine_ref, W_e1_ref, b_e1_ref, W_e2_ref, b_e2_ref,
             W_p2_ref, b_p2_ref, std_ref, mu_ref, std_out_ref, y_ref):
    e = pl.program_id(0)
    f32 = jnp.float32

    @pl.when(e == 0)
    def _():
        y_ref[...] = jnp.zeros_like(y_ref)

    x = x_ref[...]
    eh = jax.nn.relu(
        jnp.dot(x, W_e1_ref[0], preferred_element_type=f32) + b_e1_ref[0])
    eo = jnp.dot(eh, W_e2_ref[0], preferred_element_type=f32) + b_e2_ref[0]
    e_onehot = (jax.lax.broadcasted_iota(jnp.int32, (NUM_EXPERTS, 1), 0)
                == e).astype(f32)
    c = jnp.dot(combine_ref[...], e_onehot, preferred_element_type=f32)
    y_ref[...] += c * eo

    @pl.when(e == NUM_EXPERTS - 1)
    def _():
        yw = jax.nn.relu(y_ref[...])
        muv = jnp.tanh(
            jnp.dot(yw, W_p2_ref[...], preferred_element_type=f32)
            + b_p2_ref[...])
        mu_ref[...] = muv
        std_out_ref[...] = jnp.full((B, ACTION_DIM), std_ref[0, 0], f32)


def kernel(obs, std, obs_sensor, W_t, b_t, ln_g, ln_b, W_s1, b_s1, W_s2, b_s2,
           W_f1, b_f1, W_f2, b_f2, W_p1, b_p1, W_g1, b_g1, W_g2, b_g2,
           W_e1, b_e1, W_e2, b_e2, W_p2, b_p2):
    f32 = jnp.float32
    r2 = lambda v: v.reshape(1, -1)

    const = lambda shape: pl.BlockSpec(shape, lambda b: tuple(0 for _ in shape))
    x_out, combine, auxmat = pl.pallas_call(
        _k1_body,
        grid=(NB,),
        in_specs=[
            pl.BlockSpec((BB, REPR_DIM), lambda b: (b, 0)),
            pl.BlockSpec((BB, STATE_DIM), lambda b: (b, 0)),
            const((REPR_DIM, FEATURE_DIM)),
            const((1, FEATURE_DIM)),
            const((1, FEATURE_DIM)),
            const((1, FEATURE_DIM)),
            const((STATE_DIM, HIDDEN_DIM)),
            const((1, HIDDEN_DIM)),
            const((HIDDEN_DIM, FEATURE_DIM)),
            const((1, FEATURE_DIM)),
            const((FEATURE_DIM, HIDDEN_DIM)),
            const((1, HIDDEN_DIM)),
            const((HIDDEN_DIM, FEATURE_DIM)),
            const((1, FEATURE_DIM)),
            const((FEATURE_DIM, HIDDEN_DIM)),
            const((1, HIDDEN_DIM)),
            const((HIDDEN_DIM, GATE_DIM)),
            const((1, GATE_DIM)),
            const((GATE_DIM, NUM_EXPERTS)),
            const((1, NUM_EXPERTS)),
        ],
        out_specs=[
            pl.BlockSpec((BB, HIDDEN_DIM), lambda b: (b, 0)),
            pl.BlockSpec((BB, NUM_EXPERTS), lambda b: (b, 0)),
            pl.BlockSpec((8, 128), lambda b: (0, 0)),
        ],
        out_shape=[
            jax.ShapeDtypeStruct((B, HIDDEN_DIM), f32),
            jax.ShapeDtypeStruct((B, NUM_EXPERTS), f32),
            jax.ShapeDtypeStruct((8, 128), f32),
        ],
        scratch_shapes=[pltpu.VMEM((8, NUM_EXPERTS), f32)],
    )(obs, obs_sensor, W_t, r2(b_t), r2(ln_g), r2(ln_b), W_s1, r2(b_s1),
      W_s2, r2(b_s2), W_f1, r2(b_f1), W_f2, r2(b_f2), W_p1, r2(b_p1),
      W_g1, r2(b_g1), W_g2, r2(b_g2))

    mu_out, std_out = pl.pallas_call(
        _k2_body,
        grid=(NUM_EXPERTS,),
        in_specs=[
            pl.BlockSpec((B, HIDDEN_DIM), lambda e: (0, 0)),
            pl.BlockSpec((B, NUM_EXPERTS), lambda e: (0, 0)),
            pl.BlockSpec((1, HIDDEN_DIM, MOE_HIDDEN), lambda e: (e, 0, 0)),
            pl.BlockSpec((1, 1, MOE_HIDDEN), lambda e: (e, 0, 0)),
            pl.BlockSpec((1, MOE_HIDDEN, HIDDEN_DIM), lambda e: (e, 0, 0)),
            pl.BlockSpec((1, 1, HIDDEN_DIM), lambda e: (e, 0, 0)),
            pl.BlockSpec((HIDDEN_DIM, ACTION_DIM), lambda e: (0, 0)),
            pl.BlockSpec((1, ACTION_DIM), lambda e: (0, 0)),
            pl.BlockSpec(memory_space=pltpu.SMEM),
        ],
        out_specs=[
            pl.BlockSpec((B, ACTION_DIM), lambda e: (0, 0)),
            pl.BlockSpec((B, ACTION_DIM), lambda e: (0, 0)),
        ],
        out_shape=[
            jax.ShapeDtypeStruct((B, ACTION_DIM), f32),
            jax.ShapeDtypeStruct((B, ACTION_DIM), f32),
        ],
        scratch_shapes=[pltpu.VMEM((B, HIDDEN_DIM), f32)],
    )(x_out, combine, W_e1, b_e1[:, None, :], W_e2, b_e2[:, None, :],
      W_p2, r2(b_p2), std.reshape(1, 1))

    aux_loss = auxmat[0, 0]
    return (mu_out, std_out, aux_loss)


# speedup vs baseline: 1.7121x; 1.0020x over previous
import functools

import jax
import jax.numpy as jnp
from jax.experimental import pallas as pl
from jax.experimental.pallas import tpu as pltpu

B = 1024
REPR_DIM = 4096
FEATURE_DIM = 512
HIDDEN_DIM = 1024
STATE_DIM = 64
GATE_DIM = 256
MOE_HIDDEN = 256
NUM_EXPERTS = 32
TOP_K = 4
ACTION_DIM = 12

BB = 256
NB = B // BB


def _bdot(a, b):
    return jnp.dot(a.astype(jnp.bfloat16), b.astype(jnp.bfloat16),
                   preferred_element_type=jnp.float32)


def _k1_body(obs_ref, obs_sensor_ref, W_t_ref, b_t_ref, ln_g_ref, ln_b_ref,
             W_s1_ref, b_s1_ref, W_s2_ref, b_s2_ref, W_f1_ref, b_f1_ref,
             W_f2_ref, b_f2_ref, W_p1_ref, b_p1_ref, W_g1_ref, b_g1_ref,
             W_g2_ref, b_g2_ref,
             x_out_ref, combine_out_ref, aux_out_ref, acc_ref):
    b = pl.program_id(0)
    f32 = jnp.float32

    obs = obs_ref[...]
    h0 = _bdot(obs, W_t_ref[...]) + b_t_ref[...]
    mu = jnp.mean(h0, axis=-1, keepdims=True)
    var = jnp.mean((h0 - mu) ** 2, axis=-1, keepdims=True)
    h = jnp.tanh((h0 - mu) / jnp.sqrt(var + 1e-5) * ln_g_ref[...] + ln_b_ref[...])

    s1 = jax.nn.relu(_bdot(obs_sensor_ref[...], W_s1_ref[...]) + b_s1_ref[...])
    s = _bdot(s1, W_s2_ref[...]) + b_s2_ref[...]
    h = h + s

    f1 = jax.nn.relu(_bdot(h, W_f1_ref[...]) + b_f1_ref[...])
    h = _bdot(f1, W_f2_ref[...]) + b_f2_ref[...]

    x = jax.nn.relu(_bdot(h, W_p1_ref[...]) + b_p1_ref[...])
    x_out_ref[...] = x

    g1 = jax.nn.relu(
        jnp.dot(x, W_g1_ref[...], preferred_element_type=f32) + b_g1_ref[...])
    gl = jnp.dot(g1, W_g2_ref[...], preferred_element_type=f32) + b_g2_ref[...]

    m = jnp.max(gl, axis=-1, keepdims=True)
    ex = jnp.exp(gl - m)
    p = ex / jnp.sum(ex, axis=-1, keepdims=True)

    tri = (jax.lax.broadcasted_iota(jnp.int32, (NUM_EXPERTS, NUM_EXPERTS), 0)
           <= jax.lax.broadcasted_iota(jnp.int32, (NUM_EXPERTS, NUM_EXPERTS), 1)
           ).astype(f32)
    rem = p
    sel = jnp.zeros_like(p)
    for _ in range(TOP_K):
        mk = jnp.max(rem, axis=-1, keepdims=True)
        hit = (rem == mk).astype(f32)
        cs = jnp.dot(hit, tri, preferred_element_type=f32)
        first = hit * (cs == 1.0).astype(f32)
        sel = sel + first
        rem = rem - first * (rem + 1.0)
    topsum = jnp.sum(p * sel, axis=-1, keepdims=True)
    combine = p * sel / topsum
    combine_out_ref[...] = combine

    counts = jnp.sum(sel, axis=0, keepdims=True)
    psum = jnp.sum(p, axis=0, keepdims=True)
    part = jnp.concatenate([counts, psum, jnp.zeros((6, NUM_EXPERTS), f32)],
                           axis=0)

    @pl.when(b == 0)
    def _():
        acc_ref[...] = jnp.zeros_like(acc_ref)

    acc_ref[...] += part
    aux = (NUM_EXPERTS / (B * TOP_K * B)) * jnp.sum(
        acc_ref[0, :] * acc_ref[1, :])
    aux_out_ref[...] = jnp.full((8, 128), aux, f32)


def _k2_body(x_ref, combine_ref, W_e1_ref, b_e1_ref, W_e2_ref, b_e2_ref,
             W_p2_ref, b_p2_ref, std_ref, mu_ref, std_out_ref, y_ref):
    e = pl.program_id(0)
    f32 = jnp.float32

    @pl.when(e == 0)
    def _():
        y_ref[...] = jnp.zeros_like(y_ref)

    x = x_ref[...]
    eh = jax.nn.relu(_bdot(x, W_e1_ref[0]) + b_e1_ref[0])
    eo = _bdot(eh, W_e2_ref[0]) + b_e2_ref[0]
    e_onehot = (jax.lax.broadcasted_iota(jnp.int32, (NUM_EXPERTS, 1), 0)
                == e).astype(f32)
    c = jnp.dot(combine_ref[...], e_onehot, preferred_element_type=f32)
    y_ref[...] += c * eo

    @pl.when(e == NUM_EXPERTS - 1)
    def _():
        yw = jax.nn.relu(y_ref[...])
        muv = jnp.tanh(_bdot(yw, W_p2_ref[...]) + b_p2_ref[...])
        mu_ref[...] = muv
        std_out_ref[...] = jnp.full((B, ACTION_DIM), std_ref[0, 0], f32)


def kernel(obs, std, obs_sensor, W_t, b_t, ln_g, ln_b, W_s1, b_s1, W_s2, b_s2,
           W_f1, b_f1, W_f2, b_f2, W_p1, b_p1, W_g1, b_g1, W_g2, b_g2,
           W_e1, b_e1, W_e2, b_e2, W_p2, b_p2):
    f32 = jnp.float32
    r2 = lambda v: v.reshape(1, -1)

    const = lambda shape: pl.BlockSpec(shape, lambda b: tuple(0 for _ in shape))
    x_out, combine, auxmat = pl.pallas_call(
        _k1_body,
        grid=(NB,),
        in_specs=[
            pl.BlockSpec((BB, REPR_DIM), lambda b: (b, 0)),
            pl.BlockSpec((BB, STATE_DIM), lambda b: (b, 0)),
            const((REPR_DIM, FEATURE_DIM)),
            const((1, FEATURE_DIM)),
            const((1, FEATURE_DIM)),
            const((1, FEATURE_DIM)),
            const((STATE_DIM, HIDDEN_DIM)),
            const((1, HIDDEN_DIM)),
            const((HIDDEN_DIM, FEATURE_DIM)),
            const((1, FEATURE_DIM)),
            const((FEATURE_DIM, HIDDEN_DIM)),
            const((1, HIDDEN_DIM)),
            const((HIDDEN_DIM, FEATURE_DIM)),
            const((1, FEATURE_DIM)),
            const((FEATURE_DIM, HIDDEN_DIM)),
            const((1, HIDDEN_DIM)),
            const((HIDDEN_DIM, GATE_DIM)),
            const((1, GATE_DIM)),
            const((GATE_DIM, NUM_EXPERTS)),
            const((1, NUM_EXPERTS)),
        ],
        out_specs=[
            pl.BlockSpec((BB, HIDDEN_DIM), lambda b: (b, 0)),
            pl.BlockSpec((BB, NUM_EXPERTS), lambda b: (b, 0)),
            pl.BlockSpec((8, 128), lambda b: (0, 0)),
        ],
        out_shape=[
            jax.ShapeDtypeStruct((B, HIDDEN_DIM), f32),
            jax.ShapeDtypeStruct((B, NUM_EXPERTS), f32),
            jax.ShapeDtypeStruct((8, 128), f32),
        ],
        scratch_shapes=[pltpu.VMEM((8, NUM_EXPERTS), f32)],
    )(obs, obs_sensor, W_t, r2(b_t), r2(ln_g), r2(ln_b), W_s1, r2(b_s1),
      W_s2, r2(b_s2), W_f1, r2(b_f1), W_f2, r2(b_f2), W_p1, r2(b_p1),
      W_g1, r2(b_g1), W_g2, r2(b_g2))

    mu_out, std_out = pl.pallas_call(
        _k2_body,
        grid=(NUM_EXPERTS,),
        in_specs=[
            pl.BlockSpec((B, HIDDEN_DIM), lambda e: (0, 0)),
            pl.BlockSpec((B, NUM_EXPERTS), lambda e: (0, 0)),
            pl.BlockSpec((1, HIDDEN_DIM, MOE_HIDDEN), lambda e: (e, 0, 0)),
            pl.BlockSpec((1, 1, MOE_HIDDEN), lambda e: (e, 0, 0)),
            pl.BlockSpec((1, MOE_HIDDEN, HIDDEN_DIM), lambda e: (e, 0, 0)),
            pl.BlockSpec((1, 1, HIDDEN_DIM), lambda e: (e, 0, 0)),
            pl.BlockSpec((HIDDEN_DIM, ACTION_DIM), lambda e: (0, 0)),
            pl.BlockSpec((1, ACTION_DIM), lambda e: (0, 0)),
            pl.BlockSpec(memory_space=pltpu.SMEM),
        ],
        out_specs=[
            pl.BlockSpec((B, ACTION_DIM), lambda e: (0, 0)),
            pl.BlockSpec((B, ACTION_DIM), lambda e: (0, 0)),
        ],
        out_shape=[
            jax.ShapeDtypeStruct((B, ACTION_DIM), f32),
            jax.ShapeDtypeStruct((B, ACTION_DIM), f32),
        ],
        scratch_shapes=[pltpu.VMEM((B, HIDDEN_DIM), f32)],
    )(x_out, combine, W_e1, b_e1[:, None, :], W_e2, b_e2[:, None, :],
      W_p2, r2(b_p2), std.reshape(1, 1))

    aux_loss = auxmat[0, 0]
    return (mu_out, std_out, aux_loss)
